# baseline (speedup 1.0000x reference)
import functools

import jax
import jax.numpy as jnp
from jax import lax
from jax.experimental import pallas as pl
from jax.experimental.pallas import tpu as pltpu
from jax.experimental.pallas import tpu_sc as plsc

N_ROWS = 65536
C = 10
NC = 2
NS = 16
L = 16
NW = NC * NS
ROWS_PER_W = N_ROWS // NW
WORDS_PER_W = ROWS_PER_W * C
BLOCKS = ROWS_PER_W // L
NCHUNK = 4
BLOCKS_PER_CHUNK = BLOCKS // NCHUNK
WORDS_PER_CHUNK = WORDS_PER_W // NCHUNK

_mesh = plsc.VectorSubcoreMesh(core_axis_name="c", subcore_axis_name="s")


@functools.partial(
    pl.kernel,
    out_type=jax.ShapeDtypeStruct((NW * L,), jnp.float32),
    mesh=_mesh,
    compiler_params=pltpu.CompilerParams(needs_layout_passes=False),
    scratch_types=[
        pltpu.VMEM((WORDS_PER_W,), jnp.float32),
        pltpu.VMEM((ROWS_PER_W,), jnp.int32),
        pltpu.VMEM((C * L,), jnp.float32),
        pltpu.VMEM((L,), jnp.float32),
        pltpu.SemaphoreType.DMA,
        pltpu.SemaphoreType.DMA,
        pltpu.SemaphoreType.DMA,
        pltpu.SemaphoreType.DMA,
    ],
)
def _sc_loss(yp_hbm, yt_hbm, m2_hbm, out_hbm, ypv, ytv, m2v, accv, *sems):
    wid = lax.axis_index("s") * NC + lax.axis_index("c")
    row0 = wid * ROWS_PER_W
    copies = [
        pltpu.async_copy(
            yp_hbm.at[pl.ds(row0 * C + c * WORDS_PER_CHUNK, WORDS_PER_CHUNK)],
            ypv.at[pl.ds(c * WORDS_PER_CHUNK, WORDS_PER_CHUNK)],
            sems[c],
        )
        for c in range(NCHUNK)
    ]
    pltpu.sync_copy(yt_hbm.at[pl.ds(row0, ROWS_PER_W)], ytv)
    pltpu.sync_copy(m2_hbm, m2v)
    wcols = [m2v[pl.ds(j * L, L)] for j in range(C)]

    stride = lax.iota(jnp.int32, L) * C

    accs = (jnp.zeros((L,), jnp.float32),) * C
    for c in range(NCHUNK):
        copies[c].wait()

        @plsc.parallel_loop(
            c * BLOCKS_PER_CHUNK, (c + 1) * BLOCKS_PER_CHUNK, unroll=2,
            carry=accs,
        )
        def body(b, acc_in):
            yt = ytv[pl.ds(b * L, L)]
            rowbase = b * (L * C) + stride
            out = []
            for j in range(C):
                v = plsc.load_gather(ypv, [rowbase + j])
                w = wcols[j].at[yt].get(mode="promise_in_bounds")
                out.append(acc_in[j] + (v * v) * w)
            return tuple(out)

        accs = body

    total = accs[0]
    for a in accs[1:]:
        total = total + a
    accv[...] = total
    pltpu.sync_copy(accv, out_hbm.at[pl.ds(wid * L, L)])


def kernel(y_pred, y_true, ordinal_matrix):
    m2 = (ordinal_matrix * ordinal_matrix).T / y_pred.shape[0]
    m2 = jnp.pad(m2, ((0, 0), (0, L - C))).reshape(-1)
    partials = _sc_loss(y_pred.reshape(-1), y_true, m2)
    return jnp.sum(partials)


# speedup vs baseline: 2.8706x; 2.8706x over previous
import functools

import jax
import jax.numpy as jnp
from jax import lax
from jax.experimental import pallas as pl
from jax.experimental.pallas import tpu as pltpu
from jax.experimental.pallas import tpu_sc as plsc

N_ROWS = 65536
C = 10
NC = 2
NS = 16
L = 16
NW = NC * NS
ROWS_PER_W = N_ROWS // NW
BLOCKS = ROWS_PER_W // L
NCHUNK = 4
BLOCKS_PER_CHUNK = BLOCKS // NCHUNK
ROWS_PER_CHUNK = ROWS_PER_W // NCHUNK

_mesh = plsc.VectorSubcoreMesh(core_axis_name="c", subcore_axis_name="s")


@functools.partial(
    pl.kernel,
    out_type=jax.ShapeDtypeStruct((NW * L,), jnp.float32),
    mesh=_mesh,
    compiler_params=pltpu.CompilerParams(needs_layout_passes=False),
    scratch_types=[
        pltpu.VMEM((C, ROWS_PER_W), jnp.float32),
        pltpu.VMEM((ROWS_PER_W,), jnp.int32),
        pltpu.VMEM((C * L,), jnp.float32),
        pltpu.VMEM((L,), jnp.float32),
        pltpu.SemaphoreType.DMA,
        pltpu.SemaphoreType.DMA,
        pltpu.SemaphoreType.DMA,
        pltpu.SemaphoreType.DMA,
    ],
)
def _sc_loss(ypt_hbm, yt_hbm, m2_hbm, out_hbm, ypv, ytv, m2v, accv, *sems):
    wid = lax.axis_index("s") * NC + lax.axis_index("c")
    row0 = wid * ROWS_PER_W
    copies = [
        pltpu.async_copy(
            ypt_hbm.at[:, pl.ds(row0 + c * ROWS_PER_CHUNK, ROWS_PER_CHUNK)],
            ypv.at[:, pl.ds(c * ROWS_PER_CHUNK, ROWS_PER_CHUNK)],
            sems[c],
        )
        for c in range(NCHUNK)
    ]
    pltpu.sync_copy(yt_hbm.at[pl.ds(row0, ROWS_PER_W)], ytv)
    pltpu.sync_copy(m2_hbm, m2v)
    wcols = [m2v[pl.ds(j * L, L)] for j in range(C)]

    accs = (jnp.zeros((L,), jnp.float32),) * C
    for c in range(NCHUNK):
        copies[c].wait()

        @plsc.parallel_loop(
            c * BLOCKS_PER_CHUNK, (c + 1) * BLOCKS_PER_CHUNK, unroll=2,
            carry=accs,
        )
        def body(b, acc_in):
            yt = ytv[pl.ds(b * L, L)]
            out = []
            for j in range(C):
                v = ypv[j, pl.ds(b * L, L)]
                w = wcols[j].at[yt].get(mode="promise_in_bounds")
                out.append(acc_in[j] + (v * v) * w)
            return tuple(out)

        accs = body

    total = accs[0]
    for a in accs[1:]:
        total = total + a
    accv[...] = total
    pltpu.sync_copy(accv, out_hbm.at[pl.ds(wid * L, L)])


def kernel(y_pred, y_true, ordinal_matrix):
    m2 = (ordinal_matrix * ordinal_matrix).T / y_pred.shape[0]
    m2 = jnp.pad(m2, ((0, 0), (0, L - C))).reshape(-1)
    partials = _sc_loss(y_pred.T, y_true, m2)
    return jnp.sum(partials)


# speedup vs baseline: 2.9366x; 1.0230x over previous
import functools

import jax
import jax.numpy as jnp
from jax import lax
from jax.experimental import pallas as pl
from jax.experimental.pallas import tpu as pltpu
from jax.experimental.pallas import tpu_sc as plsc

N_ROWS = 65536
C = 10
NC = 2
NS = 16
L = 16
NW = NC * NS
ROWS_PER_W = N_ROWS // NW
BLOCKS = ROWS_PER_W // L

_mesh = plsc.VectorSubcoreMesh(core_axis_name="c", subcore_axis_name="s")


@functools.partial(
    pl.kernel,
    out_type=jax.ShapeDtypeStruct((NW * L,), jnp.float32),
    mesh=_mesh,
    compiler_params=pltpu.CompilerParams(needs_layout_passes=False),
    scratch_types=[
        pltpu.VMEM((C, ROWS_PER_W), jnp.float32),
        pltpu.VMEM((ROWS_PER_W,), jnp.int32),
        pltpu.VMEM((C, C), jnp.float32),
        pltpu.VMEM((L,), jnp.float32),
        pltpu.SemaphoreType.DMA,
        pltpu.SemaphoreType.DMA,
    ],
)
def _sc_loss(ypt_hbm, yt_hbm, ord_hbm, out_hbm, ypv, ytv, ordv, accv,
             yp_sem, yt_sem):
    wid = lax.axis_index("s") * NC + lax.axis_index("c")
    row0 = wid * ROWS_PER_W
    yp_copy = pltpu.async_copy(
        ypt_hbm.at[:, pl.ds(row0, ROWS_PER_W)], ypv, yp_sem
    )
    yt_copy = pltpu.async_copy(
        yt_hbm.at[pl.ds(row0, ROWS_PER_W)], ytv, yt_sem
    )
    pltpu.sync_copy(ord_hbm, ordv)

    cidx = jnp.minimum(lax.iota(jnp.int32, L), C - 1)
    wcols = []
    for j in range(C):
        m = plsc.load_gather(ordv, [cidx, jnp.full((L,), j, jnp.int32)])
        wcols.append(m * m * (1.0 / N_ROWS))

    yp_copy.wait()
    yt_copy.wait()

    @plsc.parallel_loop(0, BLOCKS, unroll=2,
                        carry=(jnp.zeros((L,), jnp.float32),) * C)
    def body(b, acc_in):
        yt = ytv[pl.ds(b * L, L)]
        out = []
        for j in range(C):
            v = ypv[j, pl.ds(b * L, L)]
            w = wcols[j].at[yt].get(mode="promise_in_bounds")
            out.append(acc_in[j] + (v * v) * w)
        return tuple(out)

    accs = body
    total = accs[0]
    for a in accs[1:]:
        total = total + a
    accv[...] = total
    pltpu.sync_copy(accv, out_hbm.at[pl.ds(wid * L, L)])


def kernel(y_pred, y_true, ordinal_matrix):
    partials = _sc_loss(y_pred.T, y_true, ordinal_matrix)
    return jnp.sum(partials)
